# Initial kernel scaffold; baseline (speedup 1.0000x reference)
import jax
import jax.numpy as jnp
from jax.experimental import pallas as pl


def kernel(message_atom, message_bond, a2b, b2a, b2revb, input_bond, W_bond, b_bond):
    raise NotImplementedError("write your pallas kernel here")


# speedup vs baseline: 1.2300x; 1.2300x over previous
import jax
import jax.numpy as jnp
from jax import lax
from jax.experimental import pallas as pl
from jax.experimental.pallas import tpu as pltpu
from jax.experimental.pallas import tpu_sc as plsc

N_ATOMS = 10000
N_BONDS = 320000
MAX_NB = 32
HID = 128
NLC = 8

NC, NS = 2, 16
NW = NC * NS

BA = 8
NBA = 40
APW = BA * NBA
PA = NW * APW

BPW = N_BONDS // NW
BB = 80
NBB = BPW // BB

MM_BLK = 2000


def _atom_body(a2b_hbm, ma_hbm, mbond_hbm, out_hbm, idx_all, rows, ma_buf, ob, sem):
    wid = lax.axis_index("s") * NC + lax.axis_index("c")
    abase = wid * APW
    pltpu.sync_copy(a2b_hbm.at[wid], idx_all)

    def batch(b, carry):
        c0 = pltpu.async_copy(mbond_hbm.at[idx_all.at[2 * b]],
                              rows.at[pl.ds(0, 128)], sem)
        c1 = pltpu.async_copy(mbond_hbm.at[idx_all.at[2 * b + 1]],
                              rows.at[pl.ds(128, 128)], sem)
        pltpu.sync_copy(ma_hbm.at[pl.ds(abase + BA * b, BA)], ma_buf)
        c0.wait()
        c1.wait()

        def atom(i, carry2):
            r0 = i * MAX_NB
            v0 = [rows[r0, pl.ds(16 * c, 16)] for c in range(NLC)]

            def red(j, acc):
                vs = [rows[r0 + j, pl.ds(16 * c, 16)] for c in range(NLC)]
                s = [acc[c] + vs[c] for c in range(NLC)]
                m = [jnp.maximum(acc[NLC + c], vs[c]) for c in range(NLC)]
                return tuple(s + m)

            acc = lax.fori_loop(1, MAX_NB, red, tuple(v0 + v0))
            for c in range(NLC):
                sl = pl.ds(16 * c, 16)
                ob[i, sl] = ma_buf[i, sl] + acc[c] * acc[NLC + c]
            return carry2

        lax.fori_loop(0, BA, atom, 0)
        pltpu.sync_copy(ob, out_hbm.at[pl.ds(abase + BA * b, BA)])
        return carry

    lax.fori_loop(0, NBA, batch, 0)


_atom_kernel = pl.kernel(
    _atom_body,
    out_type=jax.ShapeDtypeStruct((PA, HID), jnp.float32),
    mesh=plsc.VectorSubcoreMesh(core_axis_name="c", subcore_axis_name="s"),
    scratch_types=[
        pltpu.VMEM((2 * NBA, 128), jnp.int32),
        pltpu.VMEM((BA * MAX_NB, HID), jnp.float32),
        pltpu.VMEM((BA, HID), jnp.float32),
        pltpu.VMEM((BA, HID), jnp.float32),
        pltpu.SemaphoreType.DMA,
    ],
)


def _bond_body(b2a_hbm, b2revb_hbm, manew_hbm, mbond_hbm, g_hbm,
               idx_a, idx_r, rows_a, rows_r, sema, semr):
    wid = lax.axis_index("s") * NC + lax.axis_index("c")
    bbase = wid * BPW
    pltpu.sync_copy(b2a_hbm.at[wid], idx_a)
    pltpu.sync_copy(b2revb_hbm.at[wid], idx_r)

    def batch(k, carry):
        ca = pltpu.async_copy(manew_hbm.at[idx_a.at[k]], rows_a, sema)
        cr = pltpu.async_copy(mbond_hbm.at[idx_r.at[k]], rows_r, semr)
        ca.wait()
        cr.wait()

        def row(i, carry2):
            for c in range(NLC):
                sl = pl.ds(16 * c, 16)
                rows_a[i, sl] = rows_a[i, sl] - rows_r[i, sl]
            return carry2

        lax.fori_loop(0, BB, row, 0)
        pltpu.sync_copy(rows_a, g_hbm.at[pl.ds(bbase + BB * k, BB)])
        return carry

    lax.fori_loop(0, NBB, batch, 0)


_bond_kernel = pl.kernel(
    _bond_body,
    out_type=jax.ShapeDtypeStruct((N_BONDS, HID), jnp.float32),
    mesh=plsc.VectorSubcoreMesh(core_axis_name="c", subcore_axis_name="s"),
    scratch_types=[
        pltpu.VMEM((NBB, BB), jnp.int32),
        pltpu.VMEM((NBB, BB), jnp.int32),
        pltpu.VMEM((BB, HID), jnp.float32),
        pltpu.VMEM((BB, HID), jnp.float32),
        pltpu.SemaphoreType.DMA,
        pltpu.SemaphoreType.DMA,
    ],
)


def _mm_body(g_ref, in_ref, wt_ref, b_ref, o_ref):
    mm = jnp.dot(g_ref[...], wt_ref[...], preferred_element_type=jnp.float32)
    o_ref[...] = jnp.maximum(in_ref[...] + mm + b_ref[...], 0.0)


def _linear_relu(g, input_bond, wt, b2d):
    grid = N_BONDS // MM_BLK
    return pl.pallas_call(
        _mm_body,
        grid=(grid,),
        in_specs=[
            pl.BlockSpec((MM_BLK, HID), lambda i: (i, 0)),
            pl.BlockSpec((MM_BLK, HID), lambda i: (i, 0)),
            pl.BlockSpec((HID, HID), lambda i: (0, 0)),
            pl.BlockSpec((1, HID), lambda i: (0, 0)),
        ],
        out_specs=pl.BlockSpec((MM_BLK, HID), lambda i: (i, 0)),
        out_shape=jax.ShapeDtypeStruct((N_BONDS, HID), jnp.float32),
    )(g, input_bond, wt, b2d)


def kernel(message_atom, message_bond, a2b, b2a, b2revb, input_bond, W_bond, b_bond):
    a2b = a2b.astype(jnp.int32)
    b2a = b2a.astype(jnp.int32)
    b2revb = b2revb.astype(jnp.int32)

    ma_pad = jnp.pad(message_atom, ((0, PA - N_ATOMS), (0, 0)))
    a2b_pad = jnp.pad(a2b.reshape(-1), (0, (PA - N_ATOMS) * MAX_NB))
    a2b_pad = a2b_pad.reshape(NW, 2 * NBA, 128)
    b2a_r = b2a.reshape(NW, NBB, BB)
    b2revb_r = b2revb.reshape(NW, NBB, BB)

    manew_pad = _atom_kernel(a2b_pad, ma_pad, message_bond)
    g = _bond_kernel(b2a_r, b2revb_r, manew_pad, message_bond)
    mb = _linear_relu(g, input_bond, W_bond.T, b_bond.reshape(1, HID))
    return (manew_pad[:N_ATOMS], mb)
